# Initial kernel scaffold; baseline (speedup 1.0000x reference)
import jax
import jax.numpy as jnp
from jax.experimental import pallas as pl


def kernel(inputs, coords, stamp_shape_matrix):
    raise NotImplementedError("write your pallas kernel here")


# speedup vs baseline: 1.3315x; 1.3315x over previous
import functools

import jax
import jax.numpy as jnp
from jax.experimental import pallas as pl

B, D, H, W = 16, 64, 512, 512
WC = W * 5
HBLK = 256

BAR_WIDTH = W // (3 * D + 2)
GAP_WIDTH = (W - D * BAR_WIDTH) // (D + 1)
BEG = ((W - D * BAR_WIDTH) - GAP_WIDTH * (D + 1)) // 2
BAR_START0 = BEG + GAP_WIDTH
BAR_STRIDE = BAR_WIDTH + GAP_WIDTH
ROWS_PER_VALUE = H // D


def _tile_kernel(inputs_ref, inputs_t_ref, coords_ref, coords_t_ref,
                 stamp_i_ref, out_ref):
    h = pl.program_id(0)
    h0 = h * HBLK

    inp_row = inputs_ref[0]
    inp_col = inputs_t_ref[0]
    row_c = coords_t_ref[0:1, :]
    col_c = coords_ref[:, 1:2]

    f32 = jnp.float32

    hid = jax.lax.broadcasted_iota(jnp.int32, (HBLK, D), 0) + h0
    did = jax.lax.broadcasted_iota(jnp.int32, (HBLK, D), 1)

    r_t = (hid == row_c).astype(f32) * inp_row
    a_t = ((hid // ROWS_PER_VALUE) == did).astype(f32)

    dist = jnp.abs(inp_col - inp_row)
    mn = jnp.min(dist)
    mx = jnp.max(dist)
    norm = (dist - mn) / (mx - mn)
    norm_rows = jax.lax.dot(a_t, norm,
                            precision=jax.lax.Precision.HIGHEST)

    bh = jnp.clip(jnp.round(inp_row * H), 0.0, float(H)).astype(jnp.int32)
    rowmask = (hid < bh).astype(f32)

    j = jax.lax.broadcasted_iota(jnp.int32, (D, WC), 1)
    d2 = jax.lax.broadcasted_iota(jnp.int32, (D, WC), 0)
    w = j // 5
    c = j - 5 * w

    c1 = ((c == 1) & (w == col_c)).astype(f32)
    m2 = (c == 2).astype(f32) * inp_col
    u3 = ((c == 3) & ((w // ROWS_PER_VALUE) == d2)).astype(f32)
    bar_off = w - (BAR_START0 + BAR_STRIDE * d2)
    c4 = ((c == 4) & (bar_off >= 0) & (bar_off < BAR_WIDTH)).astype(f32)

    lhs = jnp.concatenate([r_t, a_t, norm_rows, rowmask], axis=1)
    rhs = jnp.concatenate([c1, m2, u3, c4], axis=0)
    acc = jax.lax.dot(lhs, rhs, precision=jax.lax.Precision.HIGHEST)

    out_ref[0] = stamp_i_ref[...] + acc


@jax.jit
def kernel(inputs, coords, stamp_shape_matrix):
    stamp_i = jnp.pad(stamp_shape_matrix, ((0, 0), (0, 0), (0, 4)))
    stamp_i = stamp_i.reshape(H, WC)
    inputs_r = inputs[:, None, :]
    inputs_c = inputs[:, :, None]
    coords_t = coords.T

    out = pl.pallas_call(
        _tile_kernel,
        grid=(H // HBLK, B),
        in_specs=[
            pl.BlockSpec((1, 1, D), lambda h, b: (b, 0, 0)),
            pl.BlockSpec((1, D, 1), lambda h, b: (b, 0, 0)),
            pl.BlockSpec((D, 2), lambda h, b: (0, 0)),
            pl.BlockSpec((2, D), lambda h, b: (0, 0)),
            pl.BlockSpec((HBLK, WC), lambda h, b: (h, 0)),
        ],
        out_specs=pl.BlockSpec((1, HBLK, WC), lambda h, b: (b, h, 0)),
        out_shape=jax.ShapeDtypeStruct((B, H, WC), jnp.float32),
    )(inputs_r, inputs_c, coords, coords_t, stamp_i)

    return out.reshape(B, H, W, 5)


# speedup vs baseline: 1.6688x; 1.2534x over previous
import functools

import jax
import jax.numpy as jnp
from jax.experimental import pallas as pl

B, D, H, W = 16, 64, 512, 512
WC = W * 5
HBLK = 256

BAR_WIDTH = W // (3 * D + 2)
GAP_WIDTH = (W - D * BAR_WIDTH) // (D + 1)
BEG = ((W - D * BAR_WIDTH) - GAP_WIDTH * (D + 1)) // 2
BAR_START0 = BEG + GAP_WIDTH
BAR_STRIDE = BAR_WIDTH + GAP_WIDTH
ROWS_PER_VALUE = H // D


def _tile_kernel(inputs_ref, inputs_t_ref, coords_ref, coords_t_ref,
                 stamp_i_ref, out_ref):
    h = pl.program_id(0)
    h0 = h * HBLK

    inp_row = inputs_ref[0]
    inp_col = inputs_t_ref[0]
    row_c = coords_t_ref[0:1, :]
    col_c = coords_ref[:, 1:2]

    f32 = jnp.float32

    hid = jax.lax.broadcasted_iota(jnp.int32, (HBLK, D), 0) + h0
    did = jax.lax.broadcasted_iota(jnp.int32, (HBLK, D), 1)

    r_t = (hid == row_c).astype(f32) * inp_row
    a_t = ((hid // ROWS_PER_VALUE) == did).astype(f32)

    dist = jnp.abs(inp_col - inp_row)
    mn = jnp.min(dist)
    mx = jnp.max(dist)
    norm = (dist - mn) / (mx - mn)
    norm_rows = jax.lax.dot(a_t, norm,
                            precision=jax.lax.Precision.DEFAULT)

    bh = jnp.clip(jnp.round(inp_row * H), 0.0, float(H)).astype(jnp.int32)
    rowmask = (hid < bh).astype(f32)

    j = jax.lax.broadcasted_iota(jnp.int32, (D, WC), 1)
    d2 = jax.lax.broadcasted_iota(jnp.int32, (D, WC), 0)
    w = j // 5
    c = j - 5 * w

    c1 = ((c == 1) & (w == col_c)).astype(f32)
    m2 = (c == 2).astype(f32) * inp_col
    u3 = ((c == 3) & ((w // ROWS_PER_VALUE) == d2)).astype(f32)
    bar_off = w - (BAR_START0 + BAR_STRIDE * d2)
    c4 = ((c == 4) & (bar_off >= 0) & (bar_off < BAR_WIDTH)).astype(f32)

    lhs = jnp.concatenate([r_t, a_t, norm_rows, rowmask], axis=1)
    rhs = jnp.concatenate([c1, m2, u3, c4], axis=0)
    acc = jax.lax.dot(lhs, rhs, precision=jax.lax.Precision.DEFAULT)

    out_ref[0] = stamp_i_ref[...] + acc


@jax.jit
def kernel(inputs, coords, stamp_shape_matrix):
    stamp_i = jnp.pad(stamp_shape_matrix, ((0, 0), (0, 0), (0, 4)))
    stamp_i = stamp_i.reshape(H, WC)
    inputs_r = inputs[:, None, :]
    inputs_c = inputs[:, :, None]
    coords_t = coords.T

    out = pl.pallas_call(
        _tile_kernel,
        grid=(H // HBLK, B),
        in_specs=[
            pl.BlockSpec((1, 1, D), lambda h, b: (b, 0, 0)),
            pl.BlockSpec((1, D, 1), lambda h, b: (b, 0, 0)),
            pl.BlockSpec((D, 2), lambda h, b: (0, 0)),
            pl.BlockSpec((2, D), lambda h, b: (0, 0)),
            pl.BlockSpec((HBLK, WC), lambda h, b: (h, 0)),
        ],
        out_specs=pl.BlockSpec((1, HBLK, WC), lambda h, b: (b, h, 0)),
        out_shape=jax.ShapeDtypeStruct((B, H, WC), jnp.float32),
    )(inputs_r, inputs_c, coords, coords_t, stamp_i)

    return out.reshape(B, H, W, 5)


# speedup vs baseline: 1.8038x; 1.0809x over previous
import functools

import jax
import jax.numpy as jnp
from jax.experimental import pallas as pl
from jax.experimental.pallas import tpu as pltpu

B, D, H, W = 16, 64, 512, 512
WC = W * 5
HBLK = 256

BAR_WIDTH = W // (3 * D + 2)
GAP_WIDTH = (W - D * BAR_WIDTH) // (D + 1)
BEG = ((W - D * BAR_WIDTH) - GAP_WIDTH * (D + 1)) // 2
BAR_START0 = BEG + GAP_WIDTH
BAR_STRIDE = BAR_WIDTH + GAP_WIDTH
ROWS_PER_VALUE = H // D


def _tile_kernel(inputs_ref, inputs_t_ref, coords_ref, coords_t_ref,
                 stamp_ref, out_ref, stamp_scr_ref):
    h = pl.program_id(0)
    b = pl.program_id(1)
    h0 = h * HBLK

    @pl.when(b == 0)
    def _():
        jg = jax.lax.broadcasted_iota(jnp.int32, (W, WC), 1)
        wg = jax.lax.broadcasted_iota(jnp.int32, (W, WC), 0)
        g0 = (jg == 5 * wg).astype(jnp.float32)
        stamp_scr_ref[...] = jax.lax.dot(
            stamp_ref[...], g0, precision=jax.lax.Precision.DEFAULT)

    inp_row = inputs_ref[0]
    inp_col = inputs_t_ref[0]
    row_c = coords_t_ref[0:1, :]
    col_c = coords_ref[:, 1:2]

    f32 = jnp.float32

    hid = jax.lax.broadcasted_iota(jnp.int32, (HBLK, D), 0) + h0
    did = jax.lax.broadcasted_iota(jnp.int32, (HBLK, D), 1)

    r_t = (hid == row_c).astype(f32) * inp_row
    a_t = ((hid // ROWS_PER_VALUE) == did).astype(f32)

    dist = jnp.abs(inp_col - inp_row)
    mn = jnp.min(dist)
    mx = jnp.max(dist)
    norm = (dist - mn) / (mx - mn)
    norm_rows = jax.lax.dot(a_t, norm,
                            precision=jax.lax.Precision.DEFAULT)

    bh = jnp.clip(jnp.round(inp_row * H), 0.0, float(H)).astype(jnp.int32)
    rowmask = (hid < bh).astype(f32)

    j = jax.lax.broadcasted_iota(jnp.int32, (D, WC), 1)
    d2 = jax.lax.broadcasted_iota(jnp.int32, (D, WC), 0)
    w = j // 5
    c = j - 5 * w

    c1 = ((c == 1) & (w == col_c)).astype(f32)
    m2 = (c == 2).astype(f32) * inp_col
    u3 = ((c == 3) & ((w // ROWS_PER_VALUE) == d2)).astype(f32)
    bar_off = w - (BAR_START0 + BAR_STRIDE * d2)
    c4 = ((c == 4) & (bar_off >= 0) & (bar_off < BAR_WIDTH)).astype(f32)

    lhs = jnp.concatenate([r_t, a_t, norm_rows, rowmask], axis=1)
    rhs = jnp.concatenate([c1, m2, u3, c4], axis=0)
    acc = jax.lax.dot(lhs, rhs, precision=jax.lax.Precision.DEFAULT)

    out_ref[0] = stamp_scr_ref[...] + acc


@jax.jit
def kernel(inputs, coords, stamp_shape_matrix):
    stamp2d = stamp_shape_matrix.reshape(H, W)
    inputs_r = inputs[:, None, :]
    inputs_c = inputs[:, :, None]
    coords_t = coords.T

    out = pl.pallas_call(
        _tile_kernel,
        grid=(H // HBLK, B),
        in_specs=[
            pl.BlockSpec((1, 1, D), lambda h, b: (b, 0, 0)),
            pl.BlockSpec((1, D, 1), lambda h, b: (b, 0, 0)),
            pl.BlockSpec((D, 2), lambda h, b: (0, 0)),
            pl.BlockSpec((2, D), lambda h, b: (0, 0)),
            pl.BlockSpec((HBLK, W), lambda h, b: (h, 0)),
        ],
        out_specs=pl.BlockSpec((1, HBLK, WC), lambda h, b: (b, h, 0)),
        out_shape=jax.ShapeDtypeStruct((B, H, WC), jnp.float32),
        scratch_shapes=[pltpu.VMEM((HBLK, WC), jnp.float32)],
    )(inputs_r, inputs_c, coords, coords_t, stamp2d)

    return out.reshape(B, H, W, 5)


# speedup vs baseline: 8.4234x; 4.6698x over previous
import functools

import jax
import jax.numpy as jnp
from jax.experimental import pallas as pl

B, D, H, W = 16, 64, 512, 512
HBLK = 256

BAR_WIDTH = W // (3 * D + 2)
GAP_WIDTH = (W - D * BAR_WIDTH) // (D + 1)
BEG = ((W - D * BAR_WIDTH) - GAP_WIDTH * (D + 1)) // 2
BAR_START0 = BEG + GAP_WIDTH
BAR_STRIDE = BAR_WIDTH + GAP_WIDTH
ROWS_PER_VALUE = H // D


def _tile_kernel(inputs_ref, inputs_t_ref, coords_ref, coords_t_ref,
                 stamp_ref, out_ref):
    h = pl.program_id(0)
    h0 = h * HBLK

    inp_row = inputs_ref[0]
    inp_col = inputs_t_ref[0]
    row_c = coords_t_ref[0:1, :]
    col_c = coords_ref[:, 1:2]

    f32 = jnp.float32
    dot = functools.partial(jax.lax.dot, precision=jax.lax.Precision.DEFAULT)

    hid = jax.lax.broadcasted_iota(jnp.int32, (HBLK, D), 0) + h0
    did = jax.lax.broadcasted_iota(jnp.int32, (HBLK, D), 1)

    r_t = (hid == row_c).astype(f32) * inp_row
    a_t = ((hid // ROWS_PER_VALUE) == did).astype(f32)

    dist = jnp.abs(inp_col - inp_row)
    mn = jnp.min(dist)
    mx = jnp.max(dist)
    norm = (dist - mn) / (mx - mn)
    norm_rows = dot(a_t, norm)

    bh = jnp.clip(jnp.round(inp_row * H), 0.0, float(H)).astype(jnp.int32)
    rowmask = (hid < bh).astype(f32)

    wv = jax.lax.broadcasted_iota(jnp.int32, (D, W), 1)
    d2 = jax.lax.broadcasted_iota(jnp.int32, (D, W), 0)

    c1m = (wv == col_c).astype(f32)
    u3m = ((wv // ROWS_PER_VALUE) == d2).astype(f32)
    bar_off = wv - (BAR_START0 + BAR_STRIDE * d2)
    c4m = ((bar_off >= 0) & (bar_off < BAR_WIDTH)).astype(f32)

    out_ref[0, 0] = stamp_ref[...]
    out_ref[0, 1] = dot(r_t, c1m)
    out_ref[0, 2] = jnp.broadcast_to(dot(a_t, inp_col), (HBLK, W))
    out_ref[0, 3] = dot(norm_rows, u3m)
    out_ref[0, 4] = dot(rowmask, c4m)


@jax.jit
def kernel(inputs, coords, stamp_shape_matrix):
    stamp2d = stamp_shape_matrix.reshape(H, W)
    inputs_r = inputs[:, None, :]
    inputs_c = inputs[:, :, None]
    coords_t = coords.T

    out = pl.pallas_call(
        _tile_kernel,
        grid=(H // HBLK, B),
        in_specs=[
            pl.BlockSpec((1, 1, D), lambda h, b: (b, 0, 0)),
            pl.BlockSpec((1, D, 1), lambda h, b: (b, 0, 0)),
            pl.BlockSpec((D, 2), lambda h, b: (0, 0)),
            pl.BlockSpec((2, D), lambda h, b: (0, 0)),
            pl.BlockSpec((HBLK, W), lambda h, b: (h, 0)),
        ],
        out_specs=pl.BlockSpec((1, 5, HBLK, W), lambda h, b: (b, 0, h, 0)),
        out_shape=jax.ShapeDtypeStruct((B, 5, H, W), jnp.float32),
    )(inputs_r, inputs_c, coords, coords_t, stamp2d)

    return jnp.transpose(out, (0, 2, 3, 1))


# speedup vs baseline: 11.0020x; 1.3061x over previous
import functools

import jax
import jax.numpy as jnp
from jax.experimental import pallas as pl

B, D, H, W = 16, 64, 512, 512
HBLK = 512

BAR_WIDTH = W // (3 * D + 2)
GAP_WIDTH = (W - D * BAR_WIDTH) // (D + 1)
BEG = ((W - D * BAR_WIDTH) - GAP_WIDTH * (D + 1)) // 2
BAR_START0 = BEG + GAP_WIDTH
BAR_STRIDE = BAR_WIDTH + GAP_WIDTH
ROWS_PER_VALUE = H // D


def _tile_kernel(inputs_ref, inputs_t_ref, coords_ref, coords_t_ref,
                 stamp_ref, out_ref):
    h = pl.program_id(0)
    h0 = h * HBLK

    inp_row = inputs_ref[0]
    inp_col = inputs_t_ref[0]
    row_c = coords_t_ref[0:1, :]
    col_c = coords_ref[:, 1:2]

    f32 = jnp.float32
    dot = functools.partial(jax.lax.dot, precision=jax.lax.Precision.DEFAULT)

    hid = jax.lax.broadcasted_iota(jnp.int32, (HBLK, D), 0) + h0
    did = jax.lax.broadcasted_iota(jnp.int32, (HBLK, D), 1)

    r_t = (hid == row_c).astype(f32) * inp_row
    a_t = ((hid // ROWS_PER_VALUE) == did).astype(f32)

    dist = jnp.abs(inp_col - inp_row)
    mn = jnp.min(dist)
    mx = jnp.max(dist)
    norm = (dist - mn) / (mx - mn)
    norm_rows = dot(a_t, norm)

    bh = jnp.clip(jnp.round(inp_row * H), 0.0, float(H)).astype(jnp.int32)
    rowmask = (hid < bh).astype(f32)

    wv = jax.lax.broadcasted_iota(jnp.int32, (D, W), 1)
    d2 = jax.lax.broadcasted_iota(jnp.int32, (D, W), 0)

    c1m = (wv == col_c).astype(f32)
    u3m = ((wv // ROWS_PER_VALUE) == d2).astype(f32)
    bar_off = wv - (BAR_START0 + BAR_STRIDE * d2)
    c4m = ((bar_off >= 0) & (bar_off < BAR_WIDTH)).astype(f32)

    out_ref[0, 0] = stamp_ref[...]
    out_ref[0, 1] = dot(r_t, c1m)
    out_ref[0, 2] = jnp.broadcast_to(dot(a_t, inp_col), (HBLK, W))
    out_ref[0, 3] = dot(norm_rows, u3m)
    out_ref[0, 4] = dot(rowmask, c4m)


@jax.jit
def kernel(inputs, coords, stamp_shape_matrix):
    stamp2d = stamp_shape_matrix.reshape(H, W)
    inputs_r = inputs[:, None, :]
    inputs_c = inputs[:, :, None]
    coords_t = coords.T

    out = pl.pallas_call(
        _tile_kernel,
        grid=(H // HBLK, B),
        in_specs=[
            pl.BlockSpec((1, 1, D), lambda h, b: (b, 0, 0)),
            pl.BlockSpec((1, D, 1), lambda h, b: (b, 0, 0)),
            pl.BlockSpec((D, 2), lambda h, b: (0, 0)),
            pl.BlockSpec((2, D), lambda h, b: (0, 0)),
            pl.BlockSpec((HBLK, W), lambda h, b: (h, 0)),
        ],
        out_specs=pl.BlockSpec((1, 5, HBLK, W), lambda h, b: (b, 0, h, 0)),
        out_shape=jax.ShapeDtypeStruct((B, 5, H, W), jnp.float32),
    )(inputs_r, inputs_c, coords, coords_t, stamp2d)

    return jnp.transpose(out, (0, 2, 3, 1))


# speedup vs baseline: 11.0149x; 1.0012x over previous
import functools

import jax
import jax.numpy as jnp
from jax.experimental import pallas as pl
from jax.experimental.pallas import tpu as pltpu

B, D, H, W = 16, 64, 512, 512
HBLK = 512

BAR_WIDTH = W // (3 * D + 2)
GAP_WIDTH = (W - D * BAR_WIDTH) // (D + 1)
BEG = ((W - D * BAR_WIDTH) - GAP_WIDTH * (D + 1)) // 2
BAR_START0 = BEG + GAP_WIDTH
BAR_STRIDE = BAR_WIDTH + GAP_WIDTH
ROWS_PER_VALUE = H // D


def _tile_kernel(inputs_ref, inputs_t_ref, coords_ref, coords_t_ref,
                 stamp_ref, out_ref):
    h = pl.program_id(0)
    h0 = h * HBLK

    inp_row = inputs_ref[0]
    inp_col = inputs_t_ref[0]
    row_c = coords_t_ref[0:1, :]
    col_c = coords_ref[:, 1:2]

    f32 = jnp.float32
    dot = functools.partial(jax.lax.dot, precision=jax.lax.Precision.DEFAULT)

    hid = jax.lax.broadcasted_iota(jnp.int32, (HBLK, D), 0) + h0
    did = jax.lax.broadcasted_iota(jnp.int32, (HBLK, D), 1)

    r_t = (hid == row_c).astype(f32) * inp_row
    a_t = ((hid // ROWS_PER_VALUE) == did).astype(f32)

    dist = jnp.abs(inp_col - inp_row)
    mn = jnp.min(dist)
    mx = jnp.max(dist)
    norm = (dist - mn) / (mx - mn)
    norm_rows = dot(a_t, norm)

    bh = jnp.clip(jnp.round(inp_row * H), 0.0, float(H)).astype(jnp.int32)
    rowmask = (hid < bh).astype(f32)

    wv = jax.lax.broadcasted_iota(jnp.int32, (D, W), 1)
    d2 = jax.lax.broadcasted_iota(jnp.int32, (D, W), 0)

    c1m = (wv == col_c).astype(f32)
    u3m = ((wv // ROWS_PER_VALUE) == d2).astype(f32)
    bar_off = wv - (BAR_START0 + BAR_STRIDE * d2)
    c4m = ((bar_off >= 0) & (bar_off < BAR_WIDTH)).astype(f32)

    out_ref[0, 0] = stamp_ref[...]
    out_ref[0, 1] = dot(r_t, c1m)
    out_ref[0, 2] = jnp.broadcast_to(dot(a_t, inp_col), (HBLK, W))
    out_ref[0, 3] = dot(norm_rows, u3m)
    out_ref[0, 4] = dot(rowmask, c4m)


@jax.jit
def kernel(inputs, coords, stamp_shape_matrix):
    stamp2d = stamp_shape_matrix.reshape(H, W)
    inputs_r = inputs[:, None, :]
    inputs_c = inputs[:, :, None]
    coords_t = coords.T

    out = pl.pallas_call(
        _tile_kernel,
        grid=(H // HBLK, B),
        in_specs=[
            pl.BlockSpec((1, 1, D), lambda h, b: (b, 0, 0)),
            pl.BlockSpec((1, D, 1), lambda h, b: (b, 0, 0)),
            pl.BlockSpec((D, 2), lambda h, b: (0, 0)),
            pl.BlockSpec((2, D), lambda h, b: (0, 0)),
            pl.BlockSpec((HBLK, W), lambda h, b: (h, 0)),
        ],
        out_specs=pl.BlockSpec((1, 5, HBLK, W), lambda h, b: (b, 0, h, 0)),
        out_shape=jax.ShapeDtypeStruct((B, 5, H, W), jnp.float32),
        compiler_params=pltpu.CompilerParams(
            dimension_semantics=("arbitrary", "parallel")),
    )(inputs_r, inputs_c, coords, coords_t, stamp2d)

    return jnp.transpose(out, (0, 2, 3, 1))
